# baseline (speedup 1.0000x reference)
import jax
import jax.numpy as jnp
from jax import lax
from jax.experimental import pallas as pl
from jax.experimental.pallas import tpu as pltpu
from jax.experimental.pallas import tpu_sc as plsc

N = 10000
E = 320000
D = 128
G = 32

NC, NS = 2, 16
NW = NC * NS
C = 128
SG = 16
S0 = 80
S1 = 80
TOT_S = NS * (S0 + S1)
EP = TOT_S * C
NP = 10240
RPT = NP // NS
R = 512
NB = NP // R

_mesh = plsc.VectorSubcoreMesh(core_axis_name="c", subcore_axis_name="s",
                               num_cores=NC, num_subcores=NS)

_f32 = jnp.float32


def _make_sc_segsum(with_deg):
    out_type = [jax.ShapeDtypeStruct((NP, D), _f32),
                jax.ShapeDtypeStruct((NP, D), _f32)]
    if with_deg:
        out_type += [jax.ShapeDtypeStruct((NP, D), _f32),
                     jax.ShapeDtypeStruct((NP, D), _f32)]
    scratch = [pltpu.VMEM((SG, C), jnp.int32),
               pltpu.VMEM((SG, C), jnp.int32),
               pltpu.VMEM((C, D), _f32),
               pltpu.VMEM((C, D), _f32),
               pltpu.VMEM_SHARED((NP, D), _f32),
               pltpu.SemaphoreType.DMA,
               pltpu.SemaphoreType.DMA,
               pltpu.SemaphoreType.DMA]

    def body(*refs):
        if with_deg:
            (x_hbm, src_hbm, dst_hbm, zacc_hbm, ones_hbm,
             acc0_out, acc1_out, deg0_out, deg1_out,
             src_v, dst_v, rows_v, rows_w, acc_sh, sem0, sem1, semS) = refs
        else:
            (x_hbm, src_hbm, dst_hbm, zacc_hbm,
             acc0_out, acc1_out,
             src_v, dst_v, rows_v, rows_w, acc_sh, sem0, sem1, semS) = refs
        cid = lax.axis_index("c")
        sid = lax.axis_index("s")
        r0 = sid * RPT
        sbase = jnp.where(cid == 0, sid * S0, NS * S0 + sid * S1)
        ngrp = jnp.where(cid == 0, S0 // SG, S1 // SG)

        def zero_acc():
            pltpu.sync_copy(zacc_hbm, rows_v)
            for k in range(RPT // C):
                pltpu.sync_copy(rows_v, acc_sh.at[pl.ds(r0 + k * C, C)])

        def copy_out(outs):
            for c in range(NC):
                @pl.when(cid == c)
                def _(c=c):
                    for k in range(RPT // C):
                        sl = pl.ds(r0 + k * C, C)
                        pltpu.sync_copy(acc_sh.at[sl], rows_v)
                        pltpu.sync_copy(rows_v, outs[c].at[sl])

        zero_acc()
        plsc.subcore_barrier()

        @pl.loop(0, ngrp)
        def _(g):
            pltpu.sync_copy(src_hbm.at[pl.ds(sbase + g * SG, SG)], src_v)
            pltpu.sync_copy(dst_hbm.at[pl.ds(sbase + g * SG, SG)], dst_v)

            @pl.loop(0, SG // 2)
            def _(h):
                j = 2 * h
                cpa = pltpu.async_copy(x_hbm.at[src_v.at[j]], rows_v, sem0)
                cpb = pltpu.async_copy(x_hbm.at[src_v.at[j + 1]], rows_w,
                                       sem1)
                cpa.wait()
                pltpu.sync_copy(rows_v, acc_sh.at[dst_v.at[j]], add=True)
                cpb.wait()
                pltpu.sync_copy(rows_w, acc_sh.at[dst_v.at[j + 1]], add=True)

        plsc.subcore_barrier()
        copy_out([acc0_out, acc1_out])

        if with_deg:
            plsc.subcore_barrier()
            zero_acc()
            plsc.subcore_barrier()
            pltpu.sync_copy(ones_hbm, rows_v)

            @pl.loop(0, ngrp)
            def _(g):
                pltpu.sync_copy(dst_hbm.at[pl.ds(sbase + g * SG, SG)], dst_v)

                @pl.loop(0, SG // 2)
                def _(h):
                    j = 2 * h
                    cpa = pltpu.async_copy(rows_v, acc_sh.at[dst_v.at[j]],
                                           semS, add=True)
                    cpb = pltpu.async_copy(rows_v,
                                           acc_sh.at[dst_v.at[j + 1]],
                                           sem1, add=True)
                    cpa.wait()
                    cpb.wait()

            plsc.subcore_barrier()
            copy_out([deg0_out, deg1_out])

    return pl.kernel(body, out_type=out_type, mesh=_mesh,
                     scratch_types=scratch)


_sc_segsum_deg = _make_sc_segsum(True)
_sc_segsum = _make_sc_segsum(False)


def _tc_block_body(accA, accB, degA, degB, x_ref, W1, W2, Wr, b, out, W12):
    i = pl.program_id(0)

    @pl.when(i == 0)
    def _():
        W12[...] = jnp.dot(W1[...], W2[...], preferred_element_type=_f32)

    d = jnp.maximum(degA[...][:, 0:1] + degB[...][:, 0:1], 1.0)
    agg = (accA[...] + accB[...]) / d
    out[...] = jnp.maximum(
        jnp.dot(agg, W12[...], preferred_element_type=_f32)
        + jnp.dot(x_ref[...], Wr[...], preferred_element_type=_f32)
        + b[...], 0.0)


_row_spec = pl.BlockSpec((R, D), lambda i: (i, 0))
_deg_spec = _row_spec


def _w_spec(r, c):
    return pl.BlockSpec((r, c), lambda i: (0, 0))


_tc_block = pl.pallas_call(
    _tc_block_body,
    grid=(NB,),
    in_specs=[_row_spec, _row_spec, _deg_spec, _deg_spec, _row_spec,
              _w_spec(D, D), _w_spec(D, D), _w_spec(D, D), _w_spec(1, D)],
    out_specs=_row_spec,
    out_shape=jax.ShapeDtypeStruct((NP, D), _f32),
    scratch_shapes=[pltpu.VMEM((D, D), _f32)],
)


def _tc_final_body(accA, accB, degA, degB, x1_ref, batch_ref,
                   W1, W2, Wr, b, Wa, ba, Wh1, bh1, Wh2, bh2,
                   out, W12, pooled):
    i = pl.program_id(0)

    @pl.when(i == 0)
    def _():
        W12[...] = jnp.dot(W1[...], W2[...], preferred_element_type=_f32)
        pooled[...] = jnp.full((G, D), -jnp.inf, _f32)

    x1 = x1_ref[...]
    d = jnp.maximum(degA[...][:, 0:1] + degB[...][:, 0:1], 1.0)
    agg = (accA[...] + accB[...]) / d
    x2 = jnp.maximum(
        jnp.dot(agg, W12[...], preferred_element_type=_f32)
        + jnp.dot(x1, Wr[...], preferred_element_type=_f32)
        + b[...], 0.0)
    wa = Wa[...]
    go = (jnp.dot(x1, wa[:D], preferred_element_type=_f32)
          + jnp.dot(x2, wa[D:], preferred_element_type=_f32) + ba[...])
    bk = batch_ref[...]
    for g in range(G):
        m = bk == float(g)

        @pl.when(jnp.any(m))
        def _():
            v = jnp.where(m, go, -jnp.inf)
            pooled[pl.ds(g, 1), :] = jnp.maximum(
                pooled[pl.ds(g, 1), :], jnp.max(v, axis=0, keepdims=True))

    @pl.when(i == NB - 1)
    def _():
        p = pooled[...]
        p = jnp.where(jnp.isfinite(p), p, 0.0)
        h = jnp.maximum(
            jnp.dot(p, Wh1[...], preferred_element_type=_f32) + bh1[...], 0.0)
        out[...] = jnp.dot(h, Wh2[...], preferred_element_type=_f32) + bh2[...]


_tc_final = pl.pallas_call(
    _tc_final_body,
    grid=(NB,),
    in_specs=[_row_spec, _row_spec, _deg_spec, _deg_spec, _row_spec,
              pl.BlockSpec((R, 1), lambda i: (i, 0)),
              _w_spec(D, D), _w_spec(D, D), _w_spec(D, D), _w_spec(1, D),
              _w_spec(2 * D, D), _w_spec(1, D),
              _w_spec(D, 64), _w_spec(1, 64), _w_spec(64, 10),
              _w_spec(1, 10)],
    out_specs=pl.BlockSpec((G, 10), lambda i: (0, 0)),
    out_shape=jax.ShapeDtypeStruct((G, 10), _f32),
    scratch_shapes=[pltpu.VMEM((D, D), _f32), pltpu.VMEM((G, D), _f32)],
)


def kernel(x, edge_index, batch, W1_0, W2_0, Wr_0, b_0,
           W1_1, W2_1, Wr_1, b_1, Wa, ba, Wh1, bh1, Wh2, bh2):
    src = edge_index[0].astype(jnp.int32)
    dst = edge_index[1].astype(jnp.int32)
    srcp = jnp.concatenate([src, jnp.zeros((EP - E,), jnp.int32)]
                           ).reshape(TOT_S, C)
    dstp = jnp.concatenate([dst, jnp.full((EP - E,), N, jnp.int32)]
                           ).reshape(TOT_S, C)
    zacc = jnp.zeros((C, D), _f32)
    ones = jnp.ones((C, D), _f32)
    xp = jnp.pad(x, ((0, NP - N), (0, 0)))
    batchf = jnp.pad(batch.astype(_f32), (0, NP - N),
                     constant_values=float(G)).reshape(NP, 1)

    acc0, acc1, deg0, deg1 = _sc_segsum_deg(xp, srcp, dstp, zacc, ones)
    x1 = _tc_block(acc0, acc1, deg0, deg1, xp, W1_0, W2_0, Wr_0,
                   b_0.reshape(1, D))
    a0, a1 = _sc_segsum(x1, srcp, dstp, zacc)
    out = _tc_final(a0, a1, deg0, deg1, x1, batchf,
                    W1_1, W2_1, Wr_1, b_1.reshape(1, D), Wa,
                    ba.reshape(1, D), Wh1, bh1.reshape(1, 64), Wh2,
                    bh2.reshape(1, 10))
    return out


# speedup vs baseline: 1.1778x; 1.1778x over previous
import jax
import jax.numpy as jnp
from jax import lax
from jax.experimental import pallas as pl
from jax.experimental.pallas import tpu as pltpu
from jax.experimental.pallas import tpu_sc as plsc

N = 10000
E = 320000
D = 128
G = 32

NC, NS = 2, 16
NW = NC * NS
C = 128
SG = 16
S0 = 144
S1 = 16
TOT_S = NS * (S0 + S1)
EP = TOT_S * C
NP = 10240
RPT = NP // NS
R = 512
NB = NP // R

_mesh = plsc.VectorSubcoreMesh(core_axis_name="c", subcore_axis_name="s",
                               num_cores=NC, num_subcores=NS)

_f32 = jnp.float32


def _make_sc_segsum(with_deg):
    out_type = [jax.ShapeDtypeStruct((NP, D), _f32),
                jax.ShapeDtypeStruct((NP, D), _f32)]
    if with_deg:
        out_type += [jax.ShapeDtypeStruct((NP, D), _f32),
                     jax.ShapeDtypeStruct((NP, D), _f32)]
    scratch = [pltpu.VMEM((SG, C), jnp.int32),
               pltpu.VMEM((SG, C), jnp.int32),
               pltpu.VMEM((C, D), _f32),
               pltpu.VMEM((C, D), _f32),
               pltpu.VMEM_SHARED((NP, D), _f32),
               pltpu.SemaphoreType.DMA,
               pltpu.SemaphoreType.DMA,
               pltpu.SemaphoreType.DMA]

    def body(*refs):
        if with_deg:
            (x_hbm, src_hbm, dst_hbm, zacc_hbm, ones_hbm,
             acc0_out, acc1_out, deg0_out, deg1_out,
             src_v, dst_v, rows_v, rows_w, acc_sh, sem0, sem1, semS) = refs
        else:
            (x_hbm, src_hbm, dst_hbm, zacc_hbm,
             acc0_out, acc1_out,
             src_v, dst_v, rows_v, rows_w, acc_sh, sem0, sem1, semS) = refs
        cid = lax.axis_index("c")
        sid = lax.axis_index("s")
        r0 = sid * RPT
        sbase = jnp.where(cid == 0, sid * S0, NS * S0 + sid * S1)
        ngrp = jnp.where(cid == 0, S0 // SG, S1 // SG)

        def zero_acc():
            pltpu.sync_copy(zacc_hbm, rows_v)
            for k in range(RPT // C):
                pltpu.sync_copy(rows_v, acc_sh.at[pl.ds(r0 + k * C, C)])

        def copy_out(outs):
            for c in range(NC):
                @pl.when(cid == c)
                def _(c=c):
                    for k in range(RPT // C):
                        sl = pl.ds(r0 + k * C, C)
                        pltpu.sync_copy(acc_sh.at[sl], rows_v)
                        pltpu.sync_copy(rows_v, outs[c].at[sl])

        zero_acc()
        plsc.subcore_barrier()

        rows = [rows_v, rows_w]
        sems = [sem0, sem1]

        @pl.loop(0, ngrp)
        def _(g):
            pltpu.sync_copy(src_hbm.at[pl.ds(sbase + g * SG, SG)], src_v)
            pltpu.sync_copy(dst_hbm.at[pl.ds(sbase + g * SG, SG)], dst_v)
            cps = [None] * SG
            cps[0] = pltpu.async_copy(x_hbm.at[src_v.at[0]], rows[0], sems[0])
            cps[1] = pltpu.async_copy(x_hbm.at[src_v.at[1]], rows[1], sems[1])
            for j in range(SG):
                b = j % 2
                cps[j].wait()
                pltpu.sync_copy(rows[b], acc_sh.at[dst_v.at[j]], add=True)
                if j + 2 < SG:
                    cps[j + 2] = pltpu.async_copy(
                        x_hbm.at[src_v.at[j + 2]], rows[b], sems[b])

        plsc.subcore_barrier()
        copy_out([acc0_out, acc1_out])

        if with_deg:
            plsc.subcore_barrier()
            zero_acc()
            plsc.subcore_barrier()
            pltpu.sync_copy(ones_hbm, rows_v)

            @pl.loop(0, ngrp)
            def _(g):
                pltpu.sync_copy(dst_hbm.at[pl.ds(sbase + g * SG, SG)], dst_v)
                cps = [pltpu.async_copy(rows_v, acc_sh.at[dst_v.at[j]],
                                        semS, add=True)
                       for j in range(SG)]
                for cp in cps:
                    cp.wait()

            plsc.subcore_barrier()
            copy_out([deg0_out, deg1_out])

    return pl.kernel(body, out_type=out_type, mesh=_mesh,
                     scratch_types=scratch)


_sc_segsum_deg = _make_sc_segsum(True)
_sc_segsum = _make_sc_segsum(False)


def _tc_block_body(accA, accB, degA, degB, x_ref, W1, W2, Wr, b, out, W12):
    i = pl.program_id(0)

    @pl.when(i == 0)
    def _():
        W12[...] = jnp.dot(W1[...], W2[...], preferred_element_type=_f32)

    d = jnp.maximum(degA[...][:, 0:1] + degB[...][:, 0:1], 1.0)
    agg = (accA[...] + accB[...]) / d
    out[...] = jnp.maximum(
        jnp.dot(agg, W12[...], preferred_element_type=_f32)
        + jnp.dot(x_ref[...], Wr[...], preferred_element_type=_f32)
        + b[...], 0.0)


_row_spec = pl.BlockSpec((R, D), lambda i: (i, 0))
_deg_spec = _row_spec


def _w_spec(r, c):
    return pl.BlockSpec((r, c), lambda i: (0, 0))


_tc_block = pl.pallas_call(
    _tc_block_body,
    grid=(NB,),
    in_specs=[_row_spec, _row_spec, _deg_spec, _deg_spec, _row_spec,
              _w_spec(D, D), _w_spec(D, D), _w_spec(D, D), _w_spec(1, D)],
    out_specs=_row_spec,
    out_shape=jax.ShapeDtypeStruct((NP, D), _f32),
    scratch_shapes=[pltpu.VMEM((D, D), _f32)],
)


def _tc_final_body(accA, accB, degA, degB, x1_ref, batch_ref,
                   W1, W2, Wr, b, Wa, ba, Wh1, bh1, Wh2, bh2,
                   out, W12, pooled):
    i = pl.program_id(0)

    @pl.when(i == 0)
    def _():
        W12[...] = jnp.dot(W1[...], W2[...], preferred_element_type=_f32)
        pooled[...] = jnp.full((G, D), -jnp.inf, _f32)

    x1 = x1_ref[...]
    d = jnp.maximum(degA[...][:, 0:1] + degB[...][:, 0:1], 1.0)
    agg = (accA[...] + accB[...]) / d
    x2 = jnp.maximum(
        jnp.dot(agg, W12[...], preferred_element_type=_f32)
        + jnp.dot(x1, Wr[...], preferred_element_type=_f32)
        + b[...], 0.0)
    wa = Wa[...]
    go = (jnp.dot(x1, wa[:D], preferred_element_type=_f32)
          + jnp.dot(x2, wa[D:], preferred_element_type=_f32) + ba[...])
    bk = batch_ref[...]
    for g in range(G):
        m = bk == float(g)

        @pl.when(jnp.any(m))
        def _():
            v = jnp.where(m, go, -jnp.inf)
            pooled[pl.ds(g, 1), :] = jnp.maximum(
                pooled[pl.ds(g, 1), :], jnp.max(v, axis=0, keepdims=True))

    @pl.when(i == NB - 1)
    def _():
        p = pooled[...]
        p = jnp.where(jnp.isfinite(p), p, 0.0)
        h = jnp.maximum(
            jnp.dot(p, Wh1[...], preferred_element_type=_f32) + bh1[...], 0.0)
        out[...] = jnp.dot(h, Wh2[...], preferred_element_type=_f32) + bh2[...]


_tc_final = pl.pallas_call(
    _tc_final_body,
    grid=(NB,),
    in_specs=[_row_spec, _row_spec, _deg_spec, _deg_spec, _row_spec,
              pl.BlockSpec((R, 1), lambda i: (i, 0)),
              _w_spec(D, D), _w_spec(D, D), _w_spec(D, D), _w_spec(1, D),
              _w_spec(2 * D, D), _w_spec(1, D),
              _w_spec(D, 64), _w_spec(1, 64), _w_spec(64, 10),
              _w_spec(1, 10)],
    out_specs=pl.BlockSpec((G, 10), lambda i: (0, 0)),
    out_shape=jax.ShapeDtypeStruct((G, 10), _f32),
    scratch_shapes=[pltpu.VMEM((D, D), _f32), pltpu.VMEM((G, D), _f32)],
)


def kernel(x, edge_index, batch, W1_0, W2_0, Wr_0, b_0,
           W1_1, W2_1, Wr_1, b_1, Wa, ba, Wh1, bh1, Wh2, bh2):
    src = edge_index[0].astype(jnp.int32)
    dst = edge_index[1].astype(jnp.int32)
    srcp = jnp.concatenate([src, jnp.zeros((EP - E,), jnp.int32)]
                           ).reshape(TOT_S, C)
    dstp = jnp.concatenate([dst, jnp.full((EP - E,), N, jnp.int32)]
                           ).reshape(TOT_S, C)
    zacc = jnp.zeros((C, D), _f32)
    ones = jnp.ones((C, D), _f32)
    xp = jnp.pad(x, ((0, NP - N), (0, 0)))
    batchf = jnp.pad(batch.astype(_f32), (0, NP - N),
                     constant_values=float(G)).reshape(NP, 1)

    acc0, acc1, deg0, deg1 = _sc_segsum_deg(xp, srcp, dstp, zacc, ones)
    x1 = _tc_block(acc0, acc1, deg0, deg1, xp, W1_0, W2_0, Wr_0,
                   b_0.reshape(1, D))
    a0, a1 = _sc_segsum(x1, srcp, dstp, zacc)
    out = _tc_final(a0, a1, deg0, deg1, x1, batchf,
                    W1_1, W2_1, Wr_1, b_1.reshape(1, D), Wa,
                    ba.reshape(1, D), Wh1, bh1.reshape(1, 64), Wh2,
                    bh2.reshape(1, 10))
    return out
